# Initial kernel scaffold; baseline (speedup 1.0000x reference)
import jax
import jax.numpy as jnp
from jax.experimental import pallas as pl


def kernel(feat0, feat1, feat2, feat3, rois, points):
    raise NotImplementedError("write your pallas kernel here")


# speedup vs baseline: 20.0529x; 20.0529x over previous
import functools

import jax
import jax.numpy as jnp
from jax import lax
from jax.experimental import pallas as pl
from jax.experimental.pallas import tpu as pltpu
from jax.experimental.pallas import tpu_sc as plsc

_STRIDES = (4.0, 8.0, 16.0, 32.0)
_HWS = ((128, 128), (64, 64), (32, 32), (16, 16))
_R = 512
_P = 128
_C = 64
_NC, _NS = 2, 16
_NW = _NC * _NS
_ROIS_PER_W = _R // _NW


def _body(tab0, tab1, tab2, tab3, rois_hbm, ptx_hbm, pty_hbm, out_hbm,
          rois_v, ptx_v, pty_v,
          idx0, idx1, idx2, idx3, w0, w1, w2, w3,
          c0, c1, c2, c3, trans, sem):
    wid = lax.axis_index("s") * _NC + lax.axis_index("c")
    roi_base = wid * _ROIS_PER_W
    pltpu.sync_copy(rois_hbm.at[pl.ds(roi_base, _ROIS_PER_W), :], rois_v)
    pltpu.sync_copy(ptx_hbm.at[pl.ds(roi_base, _ROIS_PER_W), :], ptx_v)
    pltpu.sync_copy(pty_hbm.at[pl.ds(roi_base, _ROIS_PER_W), :], pty_v)
    lane = lax.iota(jnp.int32, 16)

    for lvl, (tab, (h, w), stride) in enumerate(
            zip((tab0, tab1, tab2, tab3), _HWS, _STRIDES)):
        inv = 1.0 / stride

        def roi_body(i, _, tab=tab, h=h, w=w, inv=inv, lvl=lvl):
            row = rois_v[i, :]
            b = row[0].astype(jnp.int32)
            x1 = row[1]
            y1 = row[2]
            bw = row[3] - x1
            bh = row[4] - y1
            bbase = b * (h * w)

            for j in range(_P // 16):
                sl = pl.ds(j * 16, 16)
                px = (x1 + ptx_v[i, sl] * bw) * inv - 0.5
                py = (y1 + pty_v[i, sl] * bh) * inv - 0.5
                x0i = px.astype(jnp.int32)
                x0i = jnp.where(x0i.astype(jnp.float32) > px, x0i - 1, x0i)
                y0i = py.astype(jnp.int32)
                y0i = jnp.where(y0i.astype(jnp.float32) > py, y0i - 1, y0i)
                wx1 = px - x0i.astype(jnp.float32)
                wx0 = 1.0 - wx1
                wy1 = py - y0i.astype(jnp.float32)
                wy0 = 1.0 - wy1
                x1i = x0i + 1
                y1i = y0i + 1
                vx0 = (x0i >= 0) & (x0i <= w - 1)
                vx1 = (x1i >= 0) & (x1i <= w - 1)
                vy0 = (y0i >= 0) & (y0i <= h - 1)
                vy1 = (y1i >= 0) & (y1i <= h - 1)
                xc0 = jnp.clip(x0i, 0, w - 1)
                xc1 = jnp.clip(x1i, 0, w - 1)
                row0 = bbase + jnp.clip(y0i, 0, h - 1) * w
                row1 = bbase + jnp.clip(y1i, 0, h - 1) * w
                idx0[sl] = row0 + xc0
                idx1[sl] = row0 + xc1
                idx2[sl] = row1 + xc0
                idx3[sl] = row1 + xc1
                zero = jnp.zeros((16,), jnp.float32)
                w0[sl] = jnp.where(vy0 & vx0, wy0 * wx0, zero)
                w1[sl] = jnp.where(vy0 & vx1, wy0 * wx1, zero)
                w2[sl] = jnp.where(vy1 & vx0, wy1 * wx0, zero)
                w3[sl] = jnp.where(vy1 & vx1, wy1 * wx1, zero)

            h0 = pltpu.async_copy(tab.at[idx0], c0, sem)
            h1 = pltpu.async_copy(tab.at[idx1], c1, sem)
            h2 = pltpu.async_copy(tab.at[idx2], c2, sem)
            h3 = pltpu.async_copy(tab.at[idx3], c3, sem)
            h0.wait()
            h1.wait()
            h2.wait()
            h3.wait()

            def pv_body(j, _2):
                sl = pl.ds(j * 16, 16)
                wv0 = w0[sl]
                wv1 = w1[sl]
                wv2 = w2[sl]
                wv3 = w3[sl]
                for k in range(16):
                    p = j * 16 + k
                    a0 = wv0[k]
                    a1 = wv1[k]
                    a2 = wv2[k]
                    a3 = wv3[k]
                    for v in range(_C // 16):
                        s = pl.ds(v * 16, 16)
                        acc = (a0 * c0[p, s] + a1 * c1[p, s]
                               + a2 * c2[p, s] + a3 * c3[p, s])
                        plsc.store_scatter(
                            trans, [(v * 16 + lane) * _P + p], acc)
                return 0

            lax.fori_loop(0, _P // 16, pv_body, 0)

            obase = (roi_base + i) * (4 * _C * _P) + lvl * (_C * _P)
            pltpu.sync_copy(trans, out_hbm.at[pl.ds(obase, _C * _P)])
            return 0

        lax.fori_loop(0, _ROIS_PER_W, roi_body, 0)


@jax.jit
def _sc_extract(tab0, tab1, tab2, tab3, rois_pad, ptx, pty):
    mesh = plsc.VectorSubcoreMesh(
        core_axis_name="c", subcore_axis_name="s",
        num_cores=_NC, num_subcores=_NS)
    f32 = jnp.float32
    return pl.kernel(
        _body,
        out_type=jax.ShapeDtypeStruct((_R * 4 * _C * _P,), f32),
        mesh=mesh,
        compiler_params=pltpu.CompilerParams(
            needs_layout_passes=False, use_tc_tiling_on_sc=False),
        scratch_types=[
            pltpu.VMEM((_ROIS_PER_W, 16), f32),
            pltpu.VMEM((_ROIS_PER_W, _P), f32),
            pltpu.VMEM((_ROIS_PER_W, _P), f32),
            pltpu.VMEM((_P,), jnp.int32),
            pltpu.VMEM((_P,), jnp.int32),
            pltpu.VMEM((_P,), jnp.int32),
            pltpu.VMEM((_P,), jnp.int32),
            pltpu.VMEM((_P,), f32),
            pltpu.VMEM((_P,), f32),
            pltpu.VMEM((_P,), f32),
            pltpu.VMEM((_P,), f32),
            pltpu.VMEM((_P, _C), f32),
            pltpu.VMEM((_P, _C), f32),
            pltpu.VMEM((_P, _C), f32),
            pltpu.VMEM((_P, _C), f32),
            pltpu.VMEM((_C * _P,), f32),
            pltpu.SemaphoreType.DMA,
        ],
    )(tab0, tab1, tab2, tab3, rois_pad, ptx, pty)


def kernel(feat0, feat1, feat2, feat3, rois, points):
    tabs = [jnp.transpose(f, (0, 2, 3, 1)).reshape(-1, _C)
            for f in (feat0, feat1, feat2, feat3)]
    rois_pad = jnp.pad(rois, ((0, 0), (0, 11)))
    ptx = points[:, :, 0]
    pty = points[:, :, 1]
    out = _sc_extract(*tabs, rois_pad, ptx, pty)
    return out.reshape(_R, 4 * _C, _P)
